# baseline (speedup 1.0000x reference)
import functools

import jax
import jax.numpy as jnp
from jax import lax
from jax.experimental import pallas as pl
from jax.experimental.pallas import tpu as pltpu
from jax.experimental.pallas import tpu_sc as plsc

ROWS, COLS, NCLASS = 4096, 26, 1000
N = ROWS * COLS
NC, NS, L = 2, 16, 16
NW = NC * NS
ROWS_PER_W = N // NW
CH = 16
CHUNKS = ROWS_PER_W // CH
CHUNK_ELEMS = CH * NCLASS


def _body(idx_hbm, out_hbm, idx_v, buf0, buf1, sem0, sem1):
    cid = lax.axis_index("c")
    sid = lax.axis_index("s")
    wid = sid * NC + cid
    base_row = wid * ROWS_PER_W

    pltpu.sync_copy(idx_hbm.at[pl.ds(base_row, ROWS_PER_W)], idx_v)

    zeros16 = jnp.zeros((L,), jnp.float32)
    ones16 = jnp.ones((L,), jnp.float32)
    lane = lax.iota(jnp.int32, L)
    sems = (sem0, sem1)
    bufs = (buf0, buf1)

    def zbody(i, _):
        buf0[pl.ds(i * L, L)] = zeros16
        buf1[pl.ds(i * L, L)] = zeros16
        return 0
    lax.fori_loop(0, CHUNK_ELEMS // L, zbody, 0)

    def chunk_pos(c):
        iv = idx_v[pl.ds(c * CH, L)]
        return lane * NCLASS + iv

    def out_slice(c):
        return out_hbm.at[pl.ds((base_row + c * CH) * NCLASS, CHUNK_ELEMS)]

    def fire(c, b):
        plsc.store_scatter(bufs[b], [chunk_pos(c)], ones16)
        pltpu.make_async_copy(bufs[b], out_slice(c), sems[b]).start()

    for b in range(2):
        fire(b, b)

    def mbody(g, _):
        for b in range(2):
            c = g * 2 + b
            pltpu.make_async_copy(bufs[b], out_slice(c - 2), sems[b]).wait()
            plsc.store_scatter(bufs[b], [chunk_pos(c - 2)], zeros16)
            fire(c, b)
        return 0
    lax.fori_loop(1, CHUNKS // 2, mbody, 0)

    for b in range(2):
        c = CHUNKS - 2 + b
        pltpu.make_async_copy(bufs[b], out_slice(c), sems[b]).wait()


_onehot_sc = pl.kernel(
    _body,
    out_type=jax.ShapeDtypeStruct((N * NCLASS,), jnp.float32),
    mesh=plsc.VectorSubcoreMesh(core_axis_name="c", subcore_axis_name="s"),
    compiler_params=pltpu.CompilerParams(needs_layout_passes=False),
    scratch_types=[
        pltpu.VMEM((ROWS_PER_W,), jnp.int32),
        pltpu.VMEM((CHUNK_ELEMS,), jnp.float32),
        pltpu.VMEM((CHUNK_ELEMS,), jnp.float32),
        pltpu.SemaphoreType.DMA,
        pltpu.SemaphoreType.DMA,
    ],
)


B_TC = 128
NB_TC = ROWS // B_TC


def _tc_body(idx_ref, out_ref):
    idx = idx_ref[...]
    cls = lax.broadcasted_iota(jnp.int32, (B_TC, COLS, NCLASS), 2)
    out_ref[...] = jnp.where(idx[:, :, None] == cls, 1.0, 0.0).astype(
        jnp.float32)


_onehot_tc = pl.pallas_call(
    _tc_body,
    grid=(NB_TC,),
    in_specs=[pl.BlockSpec((B_TC, COLS), lambda i: (i, 0))],
    out_specs=pl.BlockSpec((B_TC, COLS, NCLASS), lambda i: (i, 0, 0)),
    out_shape=jax.ShapeDtypeStruct((ROWS, COLS, NCLASS), jnp.float32),
)


CB = 40
GJ = NCLASS // CB


def _tct_body(idx_ref, out_ref):
    j = pl.program_id(1)
    idxv = idx_ref[...]
    cls = j * CB + lax.broadcasted_iota(jnp.int32, (1, CB, ROWS), 1)
    out_ref[...] = jnp.where(idxv == cls, 1.0, 0.0).astype(jnp.float32)


_onehot_tct = pl.pallas_call(
    _tct_body,
    grid=(COLS, GJ),
    in_specs=[pl.BlockSpec((1, 1, ROWS), lambda r, j: (r, 0, 0))],
    out_specs=pl.BlockSpec((1, CB, ROWS), lambda r, j: (r, j, 0)),
    out_shape=jax.ShapeDtypeStruct((COLS, NCLASS, ROWS), jnp.float32),
)


@jax.jit
def kernel(indices):
    idx_t = indices.astype(jnp.int32).T.reshape(COLS, 1, ROWS)
    out_t = _onehot_tct(idx_t)
    return jnp.transpose(out_t, (2, 0, 1))


# speedup vs baseline: 2.2852x; 2.2852x over previous
import functools

import jax
import jax.numpy as jnp
from jax import lax
from jax.experimental import pallas as pl
from jax.experimental.pallas import tpu as pltpu
from jax.experimental.pallas import tpu_sc as plsc

ROWS, COLS, NCLASS = 4096, 26, 1000
N = ROWS * COLS
NC, NS, L = 2, 16, 16
NW = NC * NS
ROWS_PER_W = N // NW
CH = 16
CHUNKS = ROWS_PER_W // CH
CHUNK_ELEMS = CH * NCLASS


def _body(idx_hbm, out_hbm, idx_v, buf0, buf1, sem0, sem1):
    cid = lax.axis_index("c")
    sid = lax.axis_index("s")
    wid = sid * NC + cid
    base_row = wid * ROWS_PER_W

    pltpu.sync_copy(idx_hbm.at[pl.ds(base_row, ROWS_PER_W)], idx_v)

    zeros16 = jnp.zeros((L,), jnp.float32)
    ones16 = jnp.ones((L,), jnp.float32)
    lane = lax.iota(jnp.int32, L)
    sems = (sem0, sem1)
    bufs = (buf0, buf1)

    def zbody(i, _):
        buf0[pl.ds(i * L, L)] = zeros16
        buf1[pl.ds(i * L, L)] = zeros16
        return 0
    lax.fori_loop(0, CHUNK_ELEMS // L, zbody, 0)

    def chunk_pos(c):
        iv = idx_v[pl.ds(c * CH, L)]
        return lane * NCLASS + iv

    def out_slice(c):
        return out_hbm.at[pl.ds((base_row + c * CH) * NCLASS, CHUNK_ELEMS)]

    def fire(c, b):
        plsc.store_scatter(bufs[b], [chunk_pos(c)], ones16)
        pltpu.make_async_copy(bufs[b], out_slice(c), sems[b]).start()

    for b in range(2):
        fire(b, b)

    def mbody(g, _):
        for b in range(2):
            c = g * 2 + b
            pltpu.make_async_copy(bufs[b], out_slice(c - 2), sems[b]).wait()
            plsc.store_scatter(bufs[b], [chunk_pos(c - 2)], zeros16)
            fire(c, b)
        return 0
    lax.fori_loop(1, CHUNKS // 2, mbody, 0)

    for b in range(2):
        c = CHUNKS - 2 + b
        pltpu.make_async_copy(bufs[b], out_slice(c), sems[b]).wait()


_onehot_sc = pl.kernel(
    _body,
    out_type=jax.ShapeDtypeStruct((N * NCLASS,), jnp.float32),
    mesh=plsc.VectorSubcoreMesh(core_axis_name="c", subcore_axis_name="s"),
    compiler_params=pltpu.CompilerParams(needs_layout_passes=False),
    scratch_types=[
        pltpu.VMEM((ROWS_PER_W,), jnp.int32),
        pltpu.VMEM((CHUNK_ELEMS,), jnp.float32),
        pltpu.VMEM((CHUNK_ELEMS,), jnp.float32),
        pltpu.SemaphoreType.DMA,
        pltpu.SemaphoreType.DMA,
    ],
)


B_TC = 128
NB_TC = ROWS // B_TC


def _tc_body(idx_ref, out_ref):
    idx = idx_ref[...]
    cls = lax.broadcasted_iota(jnp.int32, (B_TC, COLS, NCLASS), 2)
    out_ref[...] = jnp.where(idx[:, :, None] == cls, 1.0, 0.0).astype(
        jnp.float32)


_onehot_tc = pl.pallas_call(
    _tc_body,
    grid=(NB_TC,),
    in_specs=[pl.BlockSpec((B_TC, COLS), lambda i: (i, 0))],
    out_specs=pl.BlockSpec((B_TC, COLS, NCLASS), lambda i: (i, 0, 0)),
    out_shape=jax.ShapeDtypeStruct((ROWS, COLS, NCLASS), jnp.float32),
)


CB = 1000
GJ = NCLASS // CB


def _tct_body(idx_ref, out_ref):
    j = pl.program_id(1)
    idxv = idx_ref[...]
    cls = j * CB + lax.broadcasted_iota(jnp.int32, (1, CB, ROWS), 1)
    out_ref[...] = jnp.where(idxv == cls, 1.0, 0.0).astype(jnp.float32)


_onehot_tct = pl.pallas_call(
    _tct_body,
    grid=(COLS, GJ),
    in_specs=[pl.BlockSpec((1, 1, ROWS), lambda r, j: (r, 0, 0))],
    out_specs=pl.BlockSpec((1, CB, ROWS), lambda r, j: (r, j, 0)),
    out_shape=jax.ShapeDtypeStruct((COLS, NCLASS, ROWS), jnp.float32),
)


@jax.jit
def kernel(indices):
    idx_t = indices.astype(jnp.int32).T.reshape(COLS, 1, ROWS)
    out_t = _onehot_tct(idx_t)
    return jnp.transpose(out_t, (2, 0, 1))


# speedup vs baseline: 2.3220x; 1.0161x over previous
import functools

import jax
import jax.numpy as jnp
from jax import lax
from jax.experimental import pallas as pl
from jax.experimental.pallas import tpu as pltpu
from jax.experimental.pallas import tpu_sc as plsc

ROWS, COLS, NCLASS = 4096, 26, 1000
N = ROWS * COLS
NC, NS, L = 2, 16, 16
NW = NC * NS
ROWS_PER_W = N // NW
CH = 16
CHUNKS = ROWS_PER_W // CH
CHUNK_ELEMS = CH * NCLASS


def _body(idx_hbm, out_hbm, idx_v, buf0, buf1, sem0, sem1):
    cid = lax.axis_index("c")
    sid = lax.axis_index("s")
    wid = sid * NC + cid
    base_row = wid * ROWS_PER_W

    pltpu.sync_copy(idx_hbm.at[pl.ds(base_row, ROWS_PER_W)], idx_v)

    zeros16 = jnp.zeros((L,), jnp.float32)
    ones16 = jnp.ones((L,), jnp.float32)
    lane = lax.iota(jnp.int32, L)
    sems = (sem0, sem1)
    bufs = (buf0, buf1)

    def zbody(i, _):
        buf0[pl.ds(i * L, L)] = zeros16
        buf1[pl.ds(i * L, L)] = zeros16
        return 0
    lax.fori_loop(0, CHUNK_ELEMS // L, zbody, 0)

    def chunk_pos(c):
        iv = idx_v[pl.ds(c * CH, L)]
        return lane * NCLASS + iv

    def out_slice(c):
        return out_hbm.at[pl.ds((base_row + c * CH) * NCLASS, CHUNK_ELEMS)]

    def fire(c, b):
        plsc.store_scatter(bufs[b], [chunk_pos(c)], ones16)
        pltpu.make_async_copy(bufs[b], out_slice(c), sems[b]).start()

    for b in range(2):
        fire(b, b)

    def mbody(g, _):
        for b in range(2):
            c = g * 2 + b
            pltpu.make_async_copy(bufs[b], out_slice(c - 2), sems[b]).wait()
            plsc.store_scatter(bufs[b], [chunk_pos(c - 2)], zeros16)
            fire(c, b)
        return 0
    lax.fori_loop(1, CHUNKS // 2, mbody, 0)

    for b in range(2):
        c = CHUNKS - 2 + b
        pltpu.make_async_copy(bufs[b], out_slice(c), sems[b]).wait()


_onehot_sc = pl.kernel(
    _body,
    out_type=jax.ShapeDtypeStruct((N * NCLASS,), jnp.float32),
    mesh=plsc.VectorSubcoreMesh(core_axis_name="c", subcore_axis_name="s"),
    compiler_params=pltpu.CompilerParams(needs_layout_passes=False),
    scratch_types=[
        pltpu.VMEM((ROWS_PER_W,), jnp.int32),
        pltpu.VMEM((CHUNK_ELEMS,), jnp.float32),
        pltpu.VMEM((CHUNK_ELEMS,), jnp.float32),
        pltpu.SemaphoreType.DMA,
        pltpu.SemaphoreType.DMA,
    ],
)


B_TC = 128
NB_TC = ROWS // B_TC


def _tc_body(idx_ref, out_ref):
    idx = idx_ref[...]
    cls = lax.broadcasted_iota(jnp.int32, (B_TC, COLS, NCLASS), 2)
    out_ref[...] = jnp.where(idx[:, :, None] == cls, 1.0, 0.0).astype(
        jnp.float32)


_onehot_tc = pl.pallas_call(
    _tc_body,
    grid=(NB_TC,),
    in_specs=[pl.BlockSpec((B_TC, COLS), lambda i: (i, 0))],
    out_specs=pl.BlockSpec((B_TC, COLS, NCLASS), lambda i: (i, 0, 0)),
    out_shape=jax.ShapeDtypeStruct((ROWS, COLS, NCLASS), jnp.float32),
)


CB = 1000
GJ = NCLASS // CB


def _tct_body(idx_ref, out_ref):
    r = pl.program_id(0)
    j = pl.program_id(1)
    idxv = idx_ref[pl.ds(r, 1), :]
    cls = j * CB + lax.broadcasted_iota(jnp.int32, (CB, ROWS), 0)
    out_ref[...] = jnp.where(idxv == cls, 1.0, 0.0).astype(jnp.float32)[None]


_onehot_tct = pl.pallas_call(
    _tct_body,
    grid=(COLS, GJ),
    in_specs=[pl.BlockSpec((COLS, ROWS), lambda r, j: (0, 0))],
    out_specs=pl.BlockSpec((1, CB, ROWS), lambda r, j: (r, j, 0)),
    out_shape=jax.ShapeDtypeStruct((COLS, NCLASS, ROWS), jnp.float32),
)


@jax.jit
def kernel(indices):
    idx_t = indices.astype(jnp.int32).T
    out_t = _onehot_tct(idx_t)
    return jnp.transpose(out_t, (2, 0, 1))
